# baseline (speedup 1.0000x reference)
import functools

import jax
import jax.numpy as jnp
from jax import lax
from jax.experimental import pallas as pl
from jax.experimental.pallas import tpu as pltpu
from jax.experimental.pallas import tpu_sc as plsc

_B = 16384
_D = 64
_NC = 2
_NS = 16
_NW = _NC * _NS
_L = 16
_NE = 2 * _B
_EPW = _NE // _NW
_NSTEP = _EPW // _L
_BPW = _B // _NW
_ENT = 1000000
_W = 512
_TAIL0 = 999936
_NREL = 1000
_NRELP = 1024
_SROWS = _NE + _L
_BIG = 2**30


def _iota():
    return lax.iota(jnp.int32, _L)


def _scan_body(ents, sids, entT, tail2, stag,
               ent_v, sid_v, buf0, buf1, rb0, rb1, sem0, sem1, wsem):
    wid = lax.axis_index("s") * _NC + lax.axis_index("c")
    base = wid * _EPW
    pltpu.sync_copy(ents.at[pl.ds(base, _EPW)], ent_v)
    pltpu.sync_copy(sids.at[pl.ds(base, _EPW)], sid_v)
    lanes = _iota()

    def window_meta(mine):
        is_tail = mine >= _TAIL0
        e0 = jnp.where(is_tail, _TAIL0, mine & ~jnp.int32(_W - 1))
        return is_tail, e0

    def fire(buf, sem, is_tail, e0):
        @pl.when(is_tail)
        def _():
            pltpu.async_copy(
                tail2, buf.at[pl.ds(0, _D), pl.ds(0, 128)], sem)

        @pl.when(jnp.logical_not(is_tail))
        def _():
            e0n = pl.multiple_of(e0, _W)
            pltpu.async_copy(
                entT.at[pl.ds(0, _D), pl.ds(e0n, _W)], buf, sem)

    def drain(buf, sem, is_tail):
        @pl.when(is_tail)
        def _():
            pltpu.make_async_copy(
                tail2, buf.at[pl.ds(0, _D), pl.ds(0, 128)], sem).wait()

        @pl.when(jnp.logical_not(is_tail))
        def _():
            pltpu.make_async_copy(
                entT.at[pl.ds(0, _D), pl.ds(0, _W)], buf, sem).wait()

    def extract(buf, rb, e16, e0, procm):
        e512 = e16 - e0
        for d in range(_D):
            dl = (jnp.full((_L,), d, jnp.int32) + lanes) & 63
            v = plsc.load_gather(buf, [dl, e512], mask=procm)
            plsc.store_scatter(rb, [lanes, dl], v, mask=procm)

    def masked_min(m, e16):
        return jnp.min(jnp.where(m, e16, _BIG))

    def resolve(buf, sem, rb, pe16, pm, e0):
        def not_done(c):
            return jnp.any(c[0])

        def wloop(c):
            pm2, _ = c
            t2, w2 = window_meta(masked_min(pm2, pe16))
            fire(buf, sem, t2, w2)
            drain(buf, sem, t2)
            pmk = jnp.logical_and(pm2, pe16 < w2 + _W)
            extract(buf, rb, pe16, w2, pmk)
            return (jnp.logical_and(pm2, jnp.logical_not(pmk)), w2)

        return lax.while_loop(not_done, wloop, (pm, e0))

    def run_step(st, buf, sem, obuf, osem, myrb, prevrb, cm):
        e16 = ent_v[pl.ds(st * _L, _L)]
        poff = jnp.maximum(st * _L - _L, 0)
        pe16 = ent_v[pl.ds(poff, _L)]
        mine = jnp.minimum(masked_min(cm, pe16), jnp.min(e16))
        is_tail, e0 = window_meta(mine)
        drain(buf, sem, is_tail)

        @pl.when(st >= 2)
        def _():
            pltpu.make_async_copy(myrb, stag.at[pl.ds(0, _L)], wsem).wait()

        pmask = jnp.logical_and(cm, pe16 < e0 + _W)
        extract(buf, prevrb, pe16, e0, pmask)
        pm = jnp.logical_and(cm, jnp.logical_not(pmask))
        pm, e0 = resolve(buf, sem, prevrb, pe16, pm, e0)

        myproc = e16 < e0 + _W
        ncm = jnp.logical_not(myproc)

        @pl.when(st + 1 < _NSTEP)
        def _():
            ne16 = ent_v[pl.ds((st + 1) * _L, _L)]
            nmine = jnp.minimum(masked_min(ncm, e16), jnp.min(ne16))
            nt, ne0 = window_meta(nmine)
            fire(obuf, osem, nt, ne0)

        @pl.when(st >= 1)
        def _():
            psid = sid_v[pl.ds(poff, _L)]
            pltpu.async_copy(prevrb, stag.at[psid], wsem)

        extract(buf, myrb, e16, e0, myproc)
        return ncm

    e16_0 = ent_v[pl.ds(0, _L)]
    t0, w0 = window_meta(jnp.min(e16_0))
    fire(buf0, sem0, t0, w0)

    def pair(i, cm):
        cm = run_step(2 * i, buf0, sem0, buf1, sem1, rb0, rb1, cm)
        cm = run_step(2 * i + 1, buf1, sem1, buf0, sem0, rb1, rb0, cm)
        return cm

    cm = lax.fori_loop(0, _NSTEP // 2, pair, jnp.full((_L,), False))

    pe16 = ent_v[pl.ds(_EPW - _L, _L)]
    t9, w9 = window_meta(masked_min(cm, pe16))
    resolve(buf1, sem1, rb1, pe16, cm, w9)
    pltpu.async_copy(rb1, stag.at[sid_v[pl.ds(_EPW - _L, _L)]], wsem)
    pltpu.make_async_copy(rb0, stag.at[pl.ds(0, _L)], wsem).wait()
    pltpu.make_async_copy(rb1, stag.at[pl.ds(0, _L)], wsem).wait()


def _reduce_body(r_idx, stag, relT, out,
                 ridx_v, h0, h1, t0, t1, rel_v, o_v, semA, semB):
    wid = lax.axis_index("s") * _NC + lax.axis_index("c")
    base = wid * _BPW
    pltpu.sync_copy(r_idx.at[pl.ds(base, _BPW)], ridx_v)
    pltpu.sync_copy(relT, rel_v)
    lanes = _iota()

    def fire(c, hv, tv, sem):
        pltpu.async_copy(stag.at[pl.ds(base + c * 64, 64)], hv, sem)
        pltpu.async_copy(stag.at[pl.ds(_B + base + c * 64, 64)], tv, sem)

    def drain(hv, tv, sem):
        pltpu.make_async_copy(stag.at[pl.ds(0, 64)], hv, sem).wait()
        pltpu.make_async_copy(stag.at[pl.ds(0, 64)], tv, sem).wait()

    def compute(c, hv, tv):
        def group(g, carry):
            off = c * 64 + g * _L
            slot = g * _L + lanes
            q16 = ridx_v[pl.ds(off, _L)]
            acc = jnp.zeros((_L,), jnp.float32)
            for d in range(_D):
                dl = (jnp.full((_L,), d, jnp.int32) + lanes) & 63
                vh = plsc.load_gather(hv, [slot, dl])
                vt = plsc.load_gather(tv, [slot, dl])
                vr = plsc.load_gather(rel_v, [dl, q16])
                acc += vh * vr * vt
            o_v[pl.ds(off, _L)] = acc
            return carry

        lax.fori_loop(0, 4, group, 0)

    fire(0, h0, t0, semA)

    def pair(i, carry):
        c0 = 2 * i
        c1 = c0 + 1
        fire(c1, h1, t1, semB)
        drain(h0, t0, semA)
        compute(c0, h0, t0)

        @pl.when(i + 1 < 4)
        def _():
            fire(c0 + 2, h0, t0, semA)

        drain(h1, t1, semB)
        compute(c1, h1, t1)
        return carry

    lax.fori_loop(0, 4, pair, 0)
    pltpu.sync_copy(o_v, out.at[pl.ds(base, _BPW)])


def kernel(h_idx, r_idx, t_idx, ent_weight, rel_weight):
    mesh = plsc.VectorSubcoreMesh(core_axis_name="c", subcore_axis_name="s")
    cp = pltpu.CompilerParams(needs_layout_passes=False)

    scan = functools.partial(
        pl.kernel,
        mesh=mesh,
        compiler_params=cp,
        out_type=jax.ShapeDtypeStruct((_SROWS, 128), jnp.float32),
        scratch_types=[
            pltpu.VMEM((_EPW,), jnp.int32),
            pltpu.VMEM((_EPW,), jnp.int32),
            pltpu.VMEM((_D, _W), jnp.float32),
            pltpu.VMEM((_D, _W), jnp.float32),
            pltpu.VMEM((_L, 128), jnp.float32),
            pltpu.VMEM((_L, 128), jnp.float32),
            pltpu.SemaphoreType.DMA,
            pltpu.SemaphoreType.DMA,
            pltpu.SemaphoreType.DMA,
        ],
    )(_scan_body)

    reduce_ = functools.partial(
        pl.kernel,
        mesh=mesh,
        compiler_params=cp,
        out_type=jax.ShapeDtypeStruct((_B,), jnp.float32),
        scratch_types=[
            pltpu.VMEM((_BPW,), jnp.int32),
            pltpu.VMEM((64, 128), jnp.float32),
            pltpu.VMEM((64, 128), jnp.float32),
            pltpu.VMEM((64, 128), jnp.float32),
            pltpu.VMEM((64, 128), jnp.float32),
            pltpu.VMEM((_D, _NRELP), jnp.float32),
            pltpu.VMEM((_BPW,), jnp.float32),
            pltpu.SemaphoreType.DMA,
            pltpu.SemaphoreType.DMA,
        ],
    )(_reduce_body)

    h32 = h_idx.astype(jnp.int32)
    t32 = t_idx.astype(jnp.int32)
    r32 = r_idx.astype(jnp.int32)
    ents = jnp.concatenate([h32, t32])
    ents_sorted, order = lax.sort(
        (ents, jnp.arange(_NE, dtype=jnp.int32)),
        num_keys=1, is_stable=False)

    entT = ent_weight.T
    tail2 = jnp.pad(
        ent_weight[_TAIL0:].T, ((0, 0), (0, 128 - (_ENT - _TAIL0))))
    relT = jnp.pad(rel_weight, ((0, _NRELP - _NREL), (0, 0))).T

    stag = scan(ents_sorted, order, entT, tail2)
    return reduce_(r32, stag, relT)


# speedup vs baseline: 1.0615x; 1.0615x over previous
import functools

import jax
import jax.numpy as jnp
from jax import lax
from jax.experimental import pallas as pl
from jax.experimental.pallas import tpu as pltpu
from jax.experimental.pallas import tpu_sc as plsc

_B = 16384
_D = 64
_NC = 2
_NS = 16
_NW = _NC * _NS
_L = 16
_NE = 2 * _B
_EPW = _NE // _NW
_NSTEP = _EPW // _L
_BPW = _B // _NW
_ENT = 1000000
_W = 768
_W1 = 512
_E0MAX = 999168
_TAIL0 = 999936
_NREL = 1000
_NRELP = 1024
_SROWS = _NE + _L
_BIG = 2**30


def _iota():
    return lax.iota(jnp.int32, _L)


def _scan_body(ents, sids, entT, tail2, stag,
               ent_v, sid_v, buf0, buf1, rb0, rb1, sem0, sem1, wsem):
    wid = lax.axis_index("s") * _NC + lax.axis_index("c")
    base = wid * _EPW
    pltpu.sync_copy(ents.at[pl.ds(base, _EPW)], ent_v)
    pltpu.sync_copy(sids.at[pl.ds(base, _EPW)], sid_v)
    lanes = _iota()

    def window_meta(e16, valid):
        mine = jnp.min(jnp.where(valid, e16, _BIG))
        is_tail = mine >= _TAIL0
        e0 = jnp.where(
            is_tail, _TAIL0,
            jnp.minimum(mine & ~jnp.int32(127), _E0MAX))
        hi = jnp.logical_and(valid, e16 < e0 + _W)
        need2 = jnp.any(jnp.logical_and(hi, e16 >= e0 + _W1))
        need2 = jnp.logical_and(need2, jnp.logical_not(is_tail))
        return is_tail, e0, need2

    def meta(st):
        e16 = ent_v[pl.ds(st * _L, _L)]
        is_tail, e0, need2 = window_meta(e16, jnp.full((_L,), True))
        return e16, is_tail, e0, need2

    def fire(buf, sem, is_tail, e0, need2):
        @pl.when(is_tail)
        def _():
            pltpu.async_copy(
                tail2, buf.at[pl.ds(0, _D), pl.ds(0, 128)], sem)

        @pl.when(jnp.logical_not(is_tail))
        def _():
            e0n = pl.multiple_of(e0, 128)
            pltpu.async_copy(
                entT.at[pl.ds(0, _D), pl.ds(e0n, _W1)],
                buf.at[pl.ds(0, _D), pl.ds(0, _W1)], sem)

        @pl.when(need2)
        def _():
            e0n = pl.multiple_of(e0, 128)
            pltpu.async_copy(
                entT.at[pl.ds(0, _D), pl.ds(e0n + _W1, _W - _W1)],
                buf.at[pl.ds(0, _D), pl.ds(_W1, _W - _W1)], sem)

    def drain(buf, sem, is_tail, need2):
        @pl.when(is_tail)
        def _():
            pltpu.make_async_copy(
                tail2, buf.at[pl.ds(0, _D), pl.ds(0, 128)], sem).wait()

        @pl.when(jnp.logical_not(is_tail))
        def _():
            pltpu.make_async_copy(
                entT.at[pl.ds(0, _D), pl.ds(0, _W1)],
                buf.at[pl.ds(0, _D), pl.ds(0, _W1)], sem).wait()

        @pl.when(need2)
        def _():
            pltpu.make_async_copy(
                entT.at[pl.ds(0, _D), pl.ds(0, _W - _W1)],
                buf.at[pl.ds(0, _D), pl.ds(_W1, _W - _W1)], sem).wait()

    def extract(buf, rb, e16, e0, procm):
        e512 = e16 - e0
        for d in range(_D):
            dl = (jnp.full((_L,), d, jnp.int32) + lanes) & 63
            v = plsc.load_gather(buf, [dl, e512], mask=procm)
            plsc.store_scatter(rb, [lanes, dl], v, mask=procm)

    def spill(buf, rb, sem, e16, unproc):
        def not_done(u):
            return jnp.any(u)

        def window(u):
            is_tail, e0, need2 = window_meta(e16, u)
            fire(buf, sem, is_tail, e0, need2)
            drain(buf, sem, is_tail, need2)
            procm = jnp.logical_and(u, e16 < e0 + _W)
            extract(buf, rb, e16, e0, procm)
            return jnp.logical_and(u, jnp.logical_not(procm))

        lax.while_loop(not_done, window, unproc)

    def do_step(st, buf, rb, sem, i, e16, is_tail, e0):
        @pl.when(i >= 1)
        def _():
            pltpu.make_async_copy(rb, stag.at[pl.ds(0, _L)], wsem).wait()

        procm = jnp.where(is_tail, jnp.full((_L,), True), e16 < (e0 + _W))
        extract(buf, rb, e16, e0, procm)
        unproc = jnp.logical_not(procm)
        spill(buf, rb, sem, e16, unproc)
        sid16 = sid_v[pl.ds(st * _L, _L)]
        pltpu.async_copy(rb, stag.at[sid16], wsem)

    e16_0, tail_0, e0_0, n2_0 = meta(0)
    fire(buf0, sem0, tail_0, e0_0, n2_0)

    def pair(i, carry):
        st0 = 2 * i
        st1 = st0 + 1
        e16a, taila, e0a, n2a = meta(st0)
        drain(buf0, sem0, taila, n2a)
        e16b, tailb, e0b, n2b = meta(st1)
        fire(buf1, sem1, tailb, e0b, n2b)
        do_step(st0, buf0, rb0, sem0, i, e16a, taila, e0a)

        drain(buf1, sem1, tailb, n2b)

        @pl.when(i + 1 < _NSTEP // 2)
        def _():
            e16c, tailc, e0c, n2c = meta(st0 + 2)
            fire(buf0, sem0, tailc, e0c, n2c)

        do_step(st1, buf1, rb1, sem1, i, e16b, tailb, e0b)
        return carry

    lax.fori_loop(0, _NSTEP // 2, pair, 0)
    pltpu.make_async_copy(rb0, stag.at[pl.ds(0, _L)], wsem).wait()
    pltpu.make_async_copy(rb1, stag.at[pl.ds(0, _L)], wsem).wait()


def _reduce_body(r_idx, stag, relT, out,
                 ridx_v, h0, h1, t0, t1, rel_v, o_v, semA, semB, semR):
    wid = lax.axis_index("s") * _NC + lax.axis_index("c")
    base = wid * _BPW
    pltpu.async_copy(r_idx.at[pl.ds(base, _BPW)], ridx_v, semR)
    pltpu.async_copy(relT, rel_v, semR)
    lanes = _iota()

    def fire(c, hv, tv, sem):
        pltpu.async_copy(stag.at[pl.ds(base + c * 64, 64)], hv, sem)
        pltpu.async_copy(stag.at[pl.ds(_B + base + c * 64, 64)], tv, sem)

    def drain(hv, tv, sem):
        pltpu.make_async_copy(stag.at[pl.ds(0, 64)], hv, sem).wait()
        pltpu.make_async_copy(stag.at[pl.ds(0, 64)], tv, sem).wait()

    def compute(c, hv, tv):
        def group(g, carry):
            off = c * 64 + g * _L
            slot = g * _L + lanes
            q16 = ridx_v[pl.ds(off, _L)]
            acc = jnp.zeros((_L,), jnp.float32)
            for d in range(_D):
                dl = (jnp.full((_L,), d, jnp.int32) + lanes) & 63
                vh = plsc.load_gather(hv, [slot, dl])
                vt = plsc.load_gather(tv, [slot, dl])
                vr = plsc.load_gather(rel_v, [dl, q16])
                acc += vh * vr * vt
            o_v[pl.ds(off, _L)] = acc
            return carry

        lax.fori_loop(0, 4, group, 0)

    fire(0, h0, t0, semA)
    pltpu.make_async_copy(r_idx.at[pl.ds(0, _BPW)], ridx_v, semR).wait()
    pltpu.make_async_copy(relT, rel_v, semR).wait()

    def pair(i, carry):
        c0 = 2 * i
        c1 = c0 + 1
        fire(c1, h1, t1, semB)
        drain(h0, t0, semA)
        compute(c0, h0, t0)

        @pl.when(i + 1 < 4)
        def _():
            fire(c0 + 2, h0, t0, semA)

        drain(h1, t1, semB)
        compute(c1, h1, t1)
        return carry

    lax.fori_loop(0, 4, pair, 0)
    pltpu.sync_copy(o_v, out.at[pl.ds(base, _BPW)])


def kernel(h_idx, r_idx, t_idx, ent_weight, rel_weight):
    mesh = plsc.VectorSubcoreMesh(core_axis_name="c", subcore_axis_name="s")
    cp = pltpu.CompilerParams(needs_layout_passes=False)

    scan = functools.partial(
        pl.kernel,
        mesh=mesh,
        compiler_params=cp,
        out_type=jax.ShapeDtypeStruct((_SROWS, 128), jnp.float32),
        scratch_types=[
            pltpu.VMEM((_EPW,), jnp.int32),
            pltpu.VMEM((_EPW,), jnp.int32),
            pltpu.VMEM((_D, _W), jnp.float32),
            pltpu.VMEM((_D, _W), jnp.float32),
            pltpu.VMEM((_L, 128), jnp.float32),
            pltpu.VMEM((_L, 128), jnp.float32),
            pltpu.SemaphoreType.DMA,
            pltpu.SemaphoreType.DMA,
            pltpu.SemaphoreType.DMA,
        ],
    )(_scan_body)

    reduce_ = functools.partial(
        pl.kernel,
        mesh=mesh,
        compiler_params=cp,
        out_type=jax.ShapeDtypeStruct((_B,), jnp.float32),
        scratch_types=[
            pltpu.VMEM((_BPW,), jnp.int32),
            pltpu.VMEM((64, 128), jnp.float32),
            pltpu.VMEM((64, 128), jnp.float32),
            pltpu.VMEM((64, 128), jnp.float32),
            pltpu.VMEM((64, 128), jnp.float32),
            pltpu.VMEM((_D, _NRELP), jnp.float32),
            pltpu.VMEM((_BPW,), jnp.float32),
            pltpu.SemaphoreType.DMA,
            pltpu.SemaphoreType.DMA,
            pltpu.SemaphoreType.DMA,
        ],
    )(_reduce_body)

    h32 = h_idx.astype(jnp.int32)
    t32 = t_idx.astype(jnp.int32)
    r32 = r_idx.astype(jnp.int32)
    ents = jnp.concatenate([h32, t32])
    ents_sorted, order = lax.sort(
        (ents, jnp.arange(_NE, dtype=jnp.int32)),
        num_keys=1, is_stable=False)

    entT = ent_weight.T
    tail2 = jnp.pad(
        ent_weight[_TAIL0:].T, ((0, 0), (0, 128 - (_ENT - _TAIL0))))
    relT = jnp.pad(rel_weight, ((0, _NRELP - _NREL), (0, 0))).T

    stag = scan(ents_sorted, order, entT, tail2)
    return reduce_(r32, stag, relT)
